# Initial kernel scaffold; baseline (speedup 1.0000x reference)
import jax
import jax.numpy as jnp
from jax.experimental import pallas as pl


def kernel(x, edge_index, edge_attr, We_emb, be_emb, c1_Wl, c1_bl, c1_Wr, c1_br, c1_We, c1_att, c1_bias, c2_Wl, c2_bl, c2_Wr, c2_br, c2_We, c2_att, c2_bias):
    raise NotImplementedError("write your pallas kernel here")


# speedup vs baseline: 6.5308x; 6.5308x over previous
import functools

import jax
import jax.numpy as jnp
from jax import lax
from jax.experimental import pallas as pl
from jax.experimental.pallas import tpu as pltpu
from jax.experimental.pallas import tpu_sc as plsc

N = 10000
E = 320000
D = 128
ED = 16
TOT = E + N
NC = 2
NS = 16
NW = NC * NS
N_PAD = 10240
ROWS_PER_TILE = N_PAD // NS

C1 = 80
NCH1 = (E // NW) // C1
M1 = E // NW

C2 = 64
NCH2 = 164
M2 = NCH2 * C2
TOTP = NW * M2
DA = D + 16


def _sc_degsum_body(d3, eaa, z32, out, d_idx, eab, acc, sem):
    cid = lax.axis_index("c")
    sid = lax.axis_index("s")
    wid = cid * NS + sid
    pltpu.sync_copy(d3.at[wid], d_idx)
    r0 = sid * ROWS_PER_TILE
    pltpu.sync_copy(z32.at[pl.ds(r0, ROWS_PER_TILE), :],
                    acc.at[pl.ds(r0, ROWS_PER_TILE), :])
    plsc.subcore_barrier()

    def chunk(j, carry):
        pltpu.async_copy(eaa.at[pl.ds(wid * M1 + j * C1, C1), :], eab, sem).wait()
        pltpu.sync_copy(eab, acc.at[d_idx.at[j]], add=True)
        return carry

    lax.fori_loop(0, NCH1, chunk, 0)
    plsc.subcore_barrier()
    pltpu.sync_copy(acc.at[pl.ds(r0, ROWS_PER_TILE), :],
                    out.at[cid, pl.ds(r0, ROWS_PER_TILE), :])


@functools.cache
def _sc_degsum():
    return pl.kernel(
        _sc_degsum_body,
        out_type=jax.ShapeDtypeStruct((NC, N_PAD, 2 * ED), jnp.float32),
        mesh=plsc.VectorSubcoreMesh(core_axis_name="c", subcore_axis_name="s",
                                    num_cores=NC, num_subcores=NS),
        compiler_params=pltpu.CompilerParams(needs_layout_passes=False, use_tc_tiling_on_sc=False),
        scratch_types=[
            pltpu.VMEM((NCH1, C1), jnp.int32),
            pltpu.VMEM((C1, 2 * ED), jnp.float32),
            pltpu.VMEM_SHARED((N_PAD, 2 * ED), jnp.float32),
            pltpu.SemaphoreType.DMA,
        ],
    )


def _sc_attn_body(s3, d3, xlt, xrt, ew, att8, z144, out,
                  si00, si01, si10, si11, di00, di01, di10, di11,
                  xlb0, xlb1, xrb0, xrb1, ewb0, ewb1,
                  attv, acc, semi0, semi1, semg0, semg1):
    cid = lax.axis_index("c")
    sid = lax.axis_index("s")
    wid = cid * NS + sid
    si = ((si00, si01), (si10, si11))
    di = ((di00, di01), (di10, di11))
    xlb = (xlb0, xlb1)
    xrb = (xrb0, xrb1)
    ewb = (ewb0, ewb1)
    semi = (semi0, semi1)
    semg = (semg0, semg1)
    RPT = N // NS
    pltpu.sync_copy(att8, attv)
    r0 = sid * RPT
    pltpu.sync_copy(z144.at[pl.ds(r0, RPT), :], acc.at[pl.ds(r0, RPT), :])
    plsc.subcore_barrier()

    def issue_idx(c, b, q):
        pltpu.async_copy(s3.at[wid, c], si[b][q], semi[b])
        pltpu.async_copy(d3.at[wid, c], di[b][q], semi[b])

    def wait_idx(c, b, q):
        pltpu.make_async_copy(s3.at[wid, c], si[b][q], semi[b]).wait()
        pltpu.make_async_copy(d3.at[wid, c], di[b][q], semi[b]).wait()

    def issue_gathers(c, b, q):
        pltpu.async_copy(xlt.at[si[b][q]], xlb[b], semg[b])
        pltpu.async_copy(xrt.at[di[b][q]], xrb[b], semg[b])
        pltpu.async_copy(ew.at[pl.ds(wid * M2 + c * C2, C2), :], ewb[b], semg[b])

    def wait_gathers(c, b, q):
        pltpu.make_async_copy(xlt.at[si[b][q]], xlb[b], semg[b]).wait()
        pltpu.make_async_copy(xrt.at[di[b][q]], xrb[b], semg[b]).wait()
        pltpu.make_async_copy(ew.at[pl.ds(wid * M2 + c * C2, C2), :],
                              ewb[b], semg[b]).wait()

    issue_idx(0, 0, 0)
    issue_idx(1, 1, 0)
    issue_idx(2, 0, 1)
    issue_idx(3, 1, 1)
    wait_idx(0, 0, 0)
    issue_gathers(0, 0, 0)
    wait_idx(1, 1, 0)
    issue_gathers(1, 1, 0)

    att_r = [attv[k, :] for k in range(8)]

    def quad(jq, carry):
        for m in range(4):
            b = m % 2
            q = m // 2
            c = jq * 4 + m
            wait_gathers(c, b, q)
            base_gid = wid * M2 + c * C2

            def group(g, gcarry):
                for u in range(8):
                    i = g * 8 + u
                    xl = [xlb[b][i, pl.ds(k * 16, 16)] for k in range(9)]
                    acc_v = None
                    for k2 in range(4):
                        xr_ab = plsc.unpack(
                            xrb[b][i, pl.ds(k2 * 32, 32)],
                            format=plsc.PackFormat.INTERLEAVED,
                            preferred_element_type=jnp.float32)
                        ew_ab = plsc.unpack(
                            ewb[b][i, pl.ds(k2 * 32, 32)],
                            format=plsc.PackFormat.INTERLEAVED,
                            preferred_element_type=jnp.float32)
                        for hh in range(2):
                            k = 2 * k2 + hh
                            v = xl[k] + xr_ab[hh] + ew_ab[hh]
                            lv = jnp.maximum(v, 0.2 * v)
                            term = att_r[k] * lv
                            acc_v = term if acc_v is None else acc_v + term
                    tot = jnp.sum(acc_v)
                    ok = (base_gid + i) < TOT
                    tot = jnp.where(ok, tot, -1e30)
                    pvec = jnp.exp(jnp.full((16,), tot))
                    for k in range(9):
                        xlb[b][i, pl.ds(k * 16, 16)] = pvec * xl[k]
                return gcarry

            lax.fori_loop(0, C2 // 8, group, 0)
            pltpu.sync_copy(xlb[b], acc.at[di[b][q]], add=True)

            @pl.when(c + 2 < NCH2)
            def _refill():
                wait_idx(c + 2, b, 1 - q)
                issue_gathers(c + 2, b, 1 - q)

                @pl.when(c + 4 < NCH2)
                def _prefetch_idx():
                    issue_idx(c + 4, b, q)
        return carry

    lax.fori_loop(0, NCH2 // 4, quad, 0)
    plsc.subcore_barrier()
    pltpu.sync_copy(acc.at[pl.ds(r0, RPT), :],
                    out.at[cid, pl.ds(r0, RPT), :])


@functools.cache
def _sc_attn():
    return pl.kernel(
        _sc_attn_body,
        out_type=jax.ShapeDtypeStruct((NC, N, DA), jnp.float32),
        mesh=plsc.VectorSubcoreMesh(core_axis_name="c", subcore_axis_name="s",
                                    num_cores=NC, num_subcores=NS),
        compiler_params=pltpu.CompilerParams(needs_layout_passes=False,
                                             use_tc_tiling_on_sc=False),
        scratch_types=(
            [pltpu.VMEM((C2,), jnp.int32)] * 8
            + [pltpu.VMEM((C2, DA), jnp.float32)] * 2
            + [pltpu.VMEM((C2, D), jnp.bfloat16)] * 4
            + [pltpu.VMEM((8, 16), jnp.float32),
               pltpu.VMEM_SHARED((N, DA), jnp.float32)]
            + [pltpu.SemaphoreType.DMA] * 4
        ),
    )


def _embed_body(x_ref, w_ref, b_ref, o_ref):
    o_ref[...] = jnp.dot(x_ref[...], w_ref[...],
                         preferred_element_type=jnp.float32) + b_ref[...]


def _tc_embed(x, w, b):
    return pl.pallas_call(
        _embed_body,
        out_shape=jax.ShapeDtypeStruct((N, D), jnp.float32),
    )(x, w, b.reshape(1, D))


def _lr_body(h_ref, wl_ref, bl_ref, wr_ref, br_ref, xl_ref, xr_ref):
    h = h_ref[...]
    xl = jnp.dot(h, wl_ref[...], preferred_element_type=jnp.float32) + bl_ref[...]
    xl_ref[:, :D] = xl
    unit = (lax.broadcasted_iota(jnp.int32, (1, 16), 1) == 0).astype(jnp.float32)
    xl_ref[:, D:] = jnp.broadcast_to(unit, (N, 16))
    xr_ref[...] = (jnp.dot(h, wr_ref[...], preferred_element_type=jnp.float32)
                   + br_ref[...]).astype(jnp.bfloat16)


def _tc_lr(h, wl, bl, wr, br):
    return pl.pallas_call(
        _lr_body,
        out_shape=[jax.ShapeDtypeStruct((N, DA), jnp.float32),
                   jax.ShapeDtypeStruct((N, D), jnp.bfloat16)],
    )(h, wl, bl.reshape(1, D), wr, br.reshape(1, D))


_EW_BR = 2592


def _ew_body(a_ref, w_ref, o_ref):
    o_ref[...] = jnp.dot(a_ref[...], w_ref[...],
                         preferred_element_type=jnp.float32).astype(jnp.bfloat16)


def _tc_ew(ea2p, we):
    return pl.pallas_call(
        _ew_body,
        grid=(TOTP // _EW_BR,),
        in_specs=[pl.BlockSpec((_EW_BR, ED), lambda i: (i, 0)),
                  pl.BlockSpec((ED, D), lambda i: (0, 0))],
        out_specs=pl.BlockSpec((_EW_BR, D), lambda i: (i, 0)),
        out_shape=jax.ShapeDtypeStruct((TOTP, D), jnp.bfloat16),
    )(ea2p, we)


def _mean_body(a_ref, o_ref):
    s = a_ref[0, :N] + a_ref[1, :N]
    deg = jnp.maximum(s[:, ED:ED + 1], 1.0)
    o_ref[...] = s[:, :ED] / deg


def _tc_mean(acc1):
    return pl.pallas_call(
        _mean_body,
        out_shape=jax.ShapeDtypeStruct((N, ED), jnp.float32),
    )(acc1)


def _final_body(a_ref, b_ref, o_ref):
    s = a_ref[0] + a_ref[1]
    g = s[:, :D] / (s[:, D:D + 1] + 1e-16) + b_ref[...]
    o_ref[...] = g * jax.nn.sigmoid(g)


def _tc_final(acc2, bias):
    return pl.pallas_call(
        _final_body,
        out_shape=jax.ShapeDtypeStruct((N, D), jnp.float32),
    )(acc2, bias.reshape(1, D))


def kernel(x, edge_index, edge_attr, We_emb, be_emb,
           c1_Wl, c1_bl, c1_Wr, c1_br, c1_We, c1_att, c1_bias,
           c2_Wl, c2_bl, c2_Wr, c2_br, c2_We, c2_att, c2_bias):
    src = edge_index[0]
    dst = edge_index[1]

    d1 = dst.reshape(NW, NCH1, C1)
    eaa = jnp.concatenate(
        [edge_attr, jnp.ones((E, 1), jnp.float32), jnp.zeros((E, 15), jnp.float32)],
        axis=1)
    loop = jnp.arange(N, dtype=jnp.int32)
    pad = jnp.zeros((TOTP - TOT,), jnp.int32)
    s_all = jnp.concatenate([src, loop, pad]).reshape(NW, NCH2, C2)
    d_all = jnp.concatenate([dst, loop, pad]).reshape(NW, NCH2, C2)
    z32 = jnp.zeros((N_PAD, 2 * ED), jnp.float32)
    z144 = jnp.zeros((N, DA), jnp.float32)

    acc1 = _sc_degsum()(d1, eaa, z32)
    ea_mean = _tc_mean(acc1)
    ea2p = jnp.concatenate(
        [edge_attr, ea_mean, jnp.zeros((TOTP - TOT, ED), jnp.float32)], axis=0)

    pidx = jnp.arange(128).reshape(4, 2, 16).transpose(0, 2, 1).reshape(128)

    h = _tc_embed(x, We_emb, be_emb)
    for (Wl, bl, Wr, br, We, att, bias) in (
            (c1_Wl, c1_bl, c1_Wr, c1_br, c1_We, c1_att, c1_bias),
            (c2_Wl, c2_bl, c2_Wr, c2_br, c2_We, c2_att, c2_bias)):
        ew = _tc_ew(ea2p, We[:, pidx])
        xl_aug, xr = _tc_lr(h, Wl, bl, Wr[:, pidx], br[pidx])
        acc2 = _sc_attn()(s_all, d_all, xl_aug, xr, ew, att.reshape(8, 16), z144)
        h = _tc_final(acc2, bias)
    return h


# speedup vs baseline: 6.7369x; 1.0316x over previous
import functools

import jax
import jax.numpy as jnp
from jax import lax
from jax.experimental import pallas as pl
from jax.experimental.pallas import tpu as pltpu
from jax.experimental.pallas import tpu_sc as plsc

N = 10000
E = 320000
D = 128
ED = 16
TOT = E + N
NC = 2
NS = 16
NW = NC * NS
N_PAD = 10240
ROWS_PER_TILE = N_PAD // NS

C1 = 80
NCH1 = (E // NW) // C1
M1 = E // NW

C2 = 48
NCH2 = 216
M2 = NCH2 * C2
TOTP = NW * M2
DA = D + 16


def _sc_degsum_body(d3, eaa, z32, out, d_idx, eab, acc, sem):
    cid = lax.axis_index("c")
    sid = lax.axis_index("s")
    wid = cid * NS + sid
    pltpu.sync_copy(d3.at[wid], d_idx)
    r0 = sid * ROWS_PER_TILE
    pltpu.sync_copy(z32.at[pl.ds(r0, ROWS_PER_TILE), :],
                    acc.at[pl.ds(r0, ROWS_PER_TILE), :])
    plsc.subcore_barrier()

    def chunk(j, carry):
        pltpu.async_copy(eaa.at[pl.ds(wid * M1 + j * C1, C1), :], eab, sem).wait()
        pltpu.sync_copy(eab, acc.at[d_idx.at[j]], add=True)
        return carry

    lax.fori_loop(0, NCH1, chunk, 0)
    plsc.subcore_barrier()
    pltpu.sync_copy(acc.at[pl.ds(r0, ROWS_PER_TILE), :],
                    out.at[cid, pl.ds(r0, ROWS_PER_TILE), :])


@functools.cache
def _sc_degsum():
    return pl.kernel(
        _sc_degsum_body,
        out_type=jax.ShapeDtypeStruct((NC, N_PAD, 2 * ED), jnp.float32),
        mesh=plsc.VectorSubcoreMesh(core_axis_name="c", subcore_axis_name="s",
                                    num_cores=NC, num_subcores=NS),
        compiler_params=pltpu.CompilerParams(needs_layout_passes=False, use_tc_tiling_on_sc=False),
        scratch_types=[
            pltpu.VMEM((NCH1, C1), jnp.int32),
            pltpu.VMEM((C1, 2 * ED), jnp.float32),
            pltpu.VMEM_SHARED((N_PAD, 2 * ED), jnp.float32),
            pltpu.SemaphoreType.DMA,
        ],
    )


def _sc_attn_body(s3, d3, xlt, xrt, ew, att8, z144, out,
                  si00, si01, si10, si11, di00, di01, di10, di11,
                  xlb0, xlb1, xrb0, xrb1, ewb0, ewb1,
                  attv, acc, semi0, semi1, semg0, semg1):
    cid = lax.axis_index("c")
    sid = lax.axis_index("s")
    wid = cid * NS + sid
    si = ((si00, si01), (si10, si11))
    di = ((di00, di01), (di10, di11))
    xlb = (xlb0, xlb1)
    xrb = (xrb0, xrb1)
    ewb = (ewb0, ewb1)
    semi = (semi0, semi1)
    semg = (semg0, semg1)
    RPT = N // NS
    pltpu.sync_copy(att8, attv)
    r0 = sid * RPT
    pltpu.sync_copy(z144.at[pl.ds(r0, RPT), :], acc.at[pl.ds(r0, RPT), :])
    plsc.subcore_barrier()

    def issue_idx(c, b, q):
        pltpu.async_copy(s3.at[wid, c], si[b][q], semi[b])
        pltpu.async_copy(d3.at[wid, c], di[b][q], semi[b])

    def wait_idx(c, b, q):
        pltpu.make_async_copy(s3.at[wid, c], si[b][q], semi[b]).wait()
        pltpu.make_async_copy(d3.at[wid, c], di[b][q], semi[b]).wait()

    def issue_gathers(c, b, q):
        pltpu.async_copy(xlt.at[si[b][q]], xlb[b], semg[b])
        pltpu.async_copy(xrt.at[di[b][q]], xrb[b], semg[b])
        pltpu.async_copy(ew.at[pl.ds(wid * M2 + c * C2, C2), :], ewb[b], semg[b])

    def wait_gathers(c, b, q):
        pltpu.make_async_copy(xlt.at[si[b][q]], xlb[b], semg[b]).wait()
        pltpu.make_async_copy(xrt.at[di[b][q]], xrb[b], semg[b]).wait()
        pltpu.make_async_copy(ew.at[pl.ds(wid * M2 + c * C2, C2), :],
                              ewb[b], semg[b]).wait()

    issue_idx(0, 0, 0)
    issue_idx(1, 1, 0)
    issue_idx(2, 0, 1)
    issue_idx(3, 1, 1)
    wait_idx(0, 0, 0)
    issue_gathers(0, 0, 0)
    wait_idx(1, 1, 0)
    issue_gathers(1, 1, 0)

    att_r = [attv[k, :] for k in range(8)]

    def quad(jq, carry):
        for m in range(4):
            b = m % 2
            q = m // 2
            c = jq * 4 + m
            wait_gathers(c, b, q)
            base_gid = wid * M2 + c * C2

            def group(g, gcarry):
                for u in range(8):
                    i = g * 8 + u
                    xl = [xlb[b][i, pl.ds(k * 16, 16)] for k in range(9)]
                    acc_v = None
                    for k2 in range(4):
                        ew_ab = plsc.unpack(
                            ewb[b][i, pl.ds(k2 * 32, 32)],
                            format=plsc.PackFormat.INTERLEAVED,
                            preferred_element_type=jnp.float32)
                        for hh in range(2):
                            k = 2 * k2 + hh
                            v = (xl[k]
                                 + xrb[b][i, pl.ds(k * 16, 16)]
                                 + ew_ab[hh])
                            lv = jnp.maximum(v, 0.2 * v)
                            term = att_r[k] * lv
                            acc_v = term if acc_v is None else acc_v + term
                    tot = jnp.sum(acc_v)
                    ok = (base_gid + i) < TOT
                    tot = jnp.where(ok, tot, -1e30)
                    pvec = jnp.exp(jnp.full((16,), tot))
                    for k in range(9):
                        xlb[b][i, pl.ds(k * 16, 16)] = pvec * xl[k]
                return gcarry

            lax.fori_loop(0, C2 // 8, group, 0)
            pltpu.sync_copy(xlb[b], acc.at[di[b][q]], add=True)

            @pl.when(c + 2 < NCH2)
            def _refill():
                wait_idx(c + 2, b, 1 - q)
                issue_gathers(c + 2, b, 1 - q)

                @pl.when(c + 4 < NCH2)
                def _prefetch_idx():
                    issue_idx(c + 4, b, q)
        return carry

    lax.fori_loop(0, NCH2 // 4, quad, 0)
    plsc.subcore_barrier()
    pltpu.sync_copy(acc.at[pl.ds(r0, RPT), :],
                    out.at[cid, pl.ds(r0, RPT), :])


@functools.cache
def _sc_attn():
    return pl.kernel(
        _sc_attn_body,
        out_type=jax.ShapeDtypeStruct((NC, N, DA), jnp.float32),
        mesh=plsc.VectorSubcoreMesh(core_axis_name="c", subcore_axis_name="s",
                                    num_cores=NC, num_subcores=NS),
        compiler_params=pltpu.CompilerParams(needs_layout_passes=False,
                                             use_tc_tiling_on_sc=False),
        scratch_types=(
            [pltpu.VMEM((C2,), jnp.int32)] * 8
            + [pltpu.VMEM((C2, DA), jnp.float32)] * 2
            + [pltpu.VMEM((C2, D), jnp.float32)] * 2
            + [pltpu.VMEM((C2, D), jnp.bfloat16)] * 2
            + [pltpu.VMEM((8, 16), jnp.float32),
               pltpu.VMEM_SHARED((N, DA), jnp.float32)]
            + [pltpu.SemaphoreType.DMA] * 4
        ),
    )


def _embed_body(x_ref, w_ref, b_ref, o_ref):
    o_ref[...] = jnp.dot(x_ref[...], w_ref[...],
                         preferred_element_type=jnp.float32) + b_ref[...]


def _tc_embed(x, w, b):
    return pl.pallas_call(
        _embed_body,
        out_shape=jax.ShapeDtypeStruct((N, D), jnp.float32),
    )(x, w, b.reshape(1, D))


def _lr_body(h_ref, wl_ref, bl_ref, wr_ref, br_ref, xl_ref, xr_ref):
    h = h_ref[...]
    xl = jnp.dot(h, wl_ref[...], preferred_element_type=jnp.float32) + bl_ref[...]
    xl_ref[:, :D] = xl
    unit = (lax.broadcasted_iota(jnp.int32, (1, 16), 1) == 0).astype(jnp.float32)
    xl_ref[:, D:] = jnp.broadcast_to(unit, (N, 16))
    xr_ref[...] = jnp.dot(h, wr_ref[...], preferred_element_type=jnp.float32) + br_ref[...]


def _tc_lr(h, wl, bl, wr, br):
    return pl.pallas_call(
        _lr_body,
        out_shape=[jax.ShapeDtypeStruct((N, DA), jnp.float32),
                   jax.ShapeDtypeStruct((N, D), jnp.float32)],
    )(h, wl, bl.reshape(1, D), wr, br.reshape(1, D))


_EW_BR = 2592


def _ew_body(a_ref, w_ref, o_ref):
    o_ref[...] = jnp.dot(a_ref[...], w_ref[...],
                         preferred_element_type=jnp.float32).astype(jnp.bfloat16)


def _tc_ew(ea2p, we):
    return pl.pallas_call(
        _ew_body,
        grid=(TOTP // _EW_BR,),
        in_specs=[pl.BlockSpec((_EW_BR, ED), lambda i: (i, 0)),
                  pl.BlockSpec((ED, D), lambda i: (0, 0))],
        out_specs=pl.BlockSpec((_EW_BR, D), lambda i: (i, 0)),
        out_shape=jax.ShapeDtypeStruct((TOTP, D), jnp.bfloat16),
    )(ea2p, we)


def _mean_body(a_ref, o_ref):
    s = a_ref[0, :N] + a_ref[1, :N]
    deg = jnp.maximum(s[:, ED:ED + 1], 1.0)
    o_ref[...] = s[:, :ED] / deg


def _tc_mean(acc1):
    return pl.pallas_call(
        _mean_body,
        out_shape=jax.ShapeDtypeStruct((N, ED), jnp.float32),
    )(acc1)


def _final_body(a_ref, b_ref, o_ref):
    s = a_ref[0] + a_ref[1]
    g = s[:, :D] / (s[:, D:D + 1] + 1e-16) + b_ref[...]
    o_ref[...] = g * jax.nn.sigmoid(g)


def _tc_final(acc2, bias):
    return pl.pallas_call(
        _final_body,
        out_shape=jax.ShapeDtypeStruct((N, D), jnp.float32),
    )(acc2, bias.reshape(1, D))


def kernel(x, edge_index, edge_attr, We_emb, be_emb,
           c1_Wl, c1_bl, c1_Wr, c1_br, c1_We, c1_att, c1_bias,
           c2_Wl, c2_bl, c2_Wr, c2_br, c2_We, c2_att, c2_bias):
    src = edge_index[0]
    dst = edge_index[1]

    d1 = dst.reshape(NW, NCH1, C1)
    eaa = jnp.concatenate(
        [edge_attr, jnp.ones((E, 1), jnp.float32), jnp.zeros((E, 15), jnp.float32)],
        axis=1)
    loop = jnp.arange(N, dtype=jnp.int32)
    pad = jnp.zeros((TOTP - TOT,), jnp.int32)
    s_all = jnp.concatenate([src, loop, pad]).reshape(NW, NCH2, C2)
    d_all = jnp.concatenate([dst, loop, pad]).reshape(NW, NCH2, C2)
    z32 = jnp.zeros((N_PAD, 2 * ED), jnp.float32)
    z144 = jnp.zeros((N, DA), jnp.float32)

    acc1 = _sc_degsum()(d1, eaa, z32)
    ea_mean = _tc_mean(acc1)
    ea2p = jnp.concatenate(
        [edge_attr, ea_mean, jnp.zeros((TOTP - TOT, ED), jnp.float32)], axis=0)

    pidx = jnp.arange(128).reshape(4, 2, 16).transpose(0, 2, 1).reshape(128)

    h = _tc_embed(x, We_emb, be_emb)
    for (Wl, bl, Wr, br, We, att, bias) in (
            (c1_Wl, c1_bl, c1_Wr, c1_br, c1_We, c1_att, c1_bias),
            (c2_Wl, c2_bl, c2_Wr, c2_br, c2_We, c2_att, c2_bias)):
        ew = _tc_ew(ea2p, We[:, pidx])
        xl_aug, xr = _tc_lr(h, Wl, bl, Wr, br)
        acc2 = _sc_attn()(s_all, d_all, xl_aug, xr, ew, att.reshape(8, 16), z144)
        h = _tc_final(acc2, bias)
    return h


# speedup vs baseline: 7.6711x; 1.1387x over previous
import functools

import jax
import jax.numpy as jnp
from jax import lax
from jax.experimental import pallas as pl
from jax.experimental.pallas import tpu as pltpu
from jax.experimental.pallas import tpu_sc as plsc

N = 10000
E = 320000
D = 128
ED = 16
TOT = E + N
NC = 2
NS = 16
NW = NC * NS
N_PAD = 10240
ROWS_PER_TILE = N_PAD // NS

C1 = 80
NCH1 = (E // NW) // C1
M1 = E // NW

C2 = 40
NCH2 = 260
M2 = NCH2 * C2
TOTP = NW * M2
DA = D + 16


def _sc_degsum_body(d3, eaa, z32, out, d_idx, eab, acc, sem):
    cid = lax.axis_index("c")
    sid = lax.axis_index("s")
    wid = cid * NS + sid
    pltpu.sync_copy(d3.at[wid], d_idx)
    r0 = sid * ROWS_PER_TILE
    pltpu.sync_copy(z32.at[pl.ds(r0, ROWS_PER_TILE), :],
                    acc.at[pl.ds(r0, ROWS_PER_TILE), :])
    plsc.subcore_barrier()

    def chunk(j, carry):
        pltpu.async_copy(eaa.at[pl.ds(wid * M1 + j * C1, C1), :], eab, sem).wait()
        pltpu.sync_copy(eab, acc.at[d_idx.at[j]], add=True)
        return carry

    lax.fori_loop(0, NCH1, chunk, 0)
    plsc.subcore_barrier()
    pltpu.sync_copy(acc.at[pl.ds(r0, ROWS_PER_TILE), :],
                    out.at[cid, pl.ds(r0, ROWS_PER_TILE), :])


@functools.cache
def _sc_degsum():
    return pl.kernel(
        _sc_degsum_body,
        out_type=jax.ShapeDtypeStruct((NC, N_PAD, 2 * ED), jnp.float32),
        mesh=plsc.VectorSubcoreMesh(core_axis_name="c", subcore_axis_name="s",
                                    num_cores=NC, num_subcores=NS),
        compiler_params=pltpu.CompilerParams(needs_layout_passes=False, use_tc_tiling_on_sc=False),
        scratch_types=[
            pltpu.VMEM((NCH1, C1), jnp.int32),
            pltpu.VMEM((C1, 2 * ED), jnp.float32),
            pltpu.VMEM_SHARED((N_PAD, 2 * ED), jnp.float32),
            pltpu.SemaphoreType.DMA,
        ],
    )


def _sc_attn_body(s3, d3, xlt, xrt, ew, att8, z144, out,
                  si00, si01, si10, si11, di00, di01, di10, di11,
                  xlb0, xlb1, xrb0, xrb1, ewb0, ewb1, outb,
                  attv, acc, semi0, semi1, semg0, semg1, sems):
    cid = lax.axis_index("c")
    sid = lax.axis_index("s")
    wid = cid * NS + sid
    si = ((si00, si01), (si10, si11))
    di = ((di00, di01), (di10, di11))
    xlb = (xlb0, xlb1)
    xrb = (xrb0, xrb1)
    ewb = (ewb0, ewb1)
    semi = (semi0, semi1)
    semg = (semg0, semg1)
    RPT = N // NS
    pltpu.sync_copy(att8, attv)
    r0 = sid * RPT
    pltpu.sync_copy(z144.at[pl.ds(r0, RPT), :], acc.at[pl.ds(r0, RPT), :])
    plsc.subcore_barrier()

    def issue_idx(c, b, q):
        pltpu.async_copy(s3.at[wid, c], si[b][q], semi[b])
        pltpu.async_copy(d3.at[wid, c], di[b][q], semi[b])

    def wait_idx(c, b, q):
        pltpu.make_async_copy(s3.at[wid, c], si[b][q], semi[b]).wait()
        pltpu.make_async_copy(d3.at[wid, c], di[b][q], semi[b]).wait()

    def issue_gathers(c, b, q):
        pltpu.async_copy(xlt.at[si[b][q]], xlb[b], semg[b])
        pltpu.async_copy(xrt.at[di[b][q]], xrb[b], semg[b])
        pltpu.async_copy(ew.at[pl.ds(wid * M2 + c * C2, C2), :], ewb[b], semg[b])

    def wait_gathers(c, b, q):
        pltpu.make_async_copy(xlt.at[si[b][q]], xlb[b], semg[b]).wait()
        pltpu.make_async_copy(xrt.at[di[b][q]], xrb[b], semg[b]).wait()
        pltpu.make_async_copy(ew.at[pl.ds(wid * M2 + c * C2, C2), :],
                              ewb[b], semg[b]).wait()

    def wait_scatter(b, q):
        pltpu.make_async_copy(outb, acc.at[di[b][q]], sems).wait()

    issue_idx(0, 0, 0)
    issue_idx(1, 1, 0)
    issue_idx(2, 0, 1)
    issue_idx(3, 1, 1)
    wait_idx(0, 0, 0)
    issue_gathers(0, 0, 0)
    wait_idx(1, 1, 0)
    issue_gathers(1, 1, 0)

    att_r = [attv[k, :] for k in range(8)]

    def quad(jq, carry):
        for m in range(4):
            b = m % 2
            q = m // 2
            pb = (m + 1) % 2
            pq = (1, 0, 0, 1)[m]
            c = jq * 4 + m
            wait_gathers(c, b, q)

            @pl.when(c > 0)
            def _drain_prev():
                wait_scatter(pb, pq)

                @pl.when(c + 3 < NCH2)
                def _prefetch_idx():
                    issue_idx(c + 3, pb, pq)

            base_gid = wid * M2 + c * C2

            def group(g, gcarry):
                for u in range(8):
                    i = g * 8 + u
                    xl = [xlb[b][i, pl.ds(k * 16, 16)] for k in range(9)]
                    acc_v = None
                    for k in range(8):
                        v = (xl[k]
                             + xrb[b][i, pl.ds(k * 16, 16)]
                             + ewb[b][i, pl.ds(k * 16, 16)])
                        lv = jnp.maximum(v, 0.2 * v)
                        term = att_r[k] * lv
                        acc_v = term if acc_v is None else acc_v + term
                    tot = jnp.sum(acc_v)
                    ok = (base_gid + i) < TOT
                    tot = jnp.where(ok, tot, -1e30)
                    pvec = jnp.exp(jnp.full((16,), tot))
                    for k in range(9):
                        outb[i, pl.ds(k * 16, 16)] = pvec * xl[k]
                return gcarry

            lax.fori_loop(0, C2 // 8, group, 0)
            pltpu.async_copy(outb, acc.at[di[b][q]], sems, add=True)

            @pl.when(c + 2 < NCH2)
            def _refill():
                wait_idx(c + 2, b, 1 - q)
                issue_gathers(c + 2, b, 1 - q)
        return carry

    lax.fori_loop(0, NCH2 // 4, quad, 0)
    wait_scatter((NCH2 - 1) % 2, ((NCH2 - 1) // 2) % 2)
    plsc.subcore_barrier()
    pltpu.sync_copy(acc.at[pl.ds(r0, RPT), :],
                    out.at[cid, pl.ds(r0, RPT), :])


@functools.cache
def _sc_attn():
    return pl.kernel(
        _sc_attn_body,
        out_type=jax.ShapeDtypeStruct((NC, N, DA), jnp.float32),
        mesh=plsc.VectorSubcoreMesh(core_axis_name="c", subcore_axis_name="s",
                                    num_cores=NC, num_subcores=NS),
        compiler_params=pltpu.CompilerParams(needs_layout_passes=False,
                                             use_tc_tiling_on_sc=False),
        scratch_types=(
            [pltpu.VMEM((C2,), jnp.int32)] * 8
            + [pltpu.VMEM((C2, DA), jnp.float32)] * 2
            + [pltpu.VMEM((C2, D), jnp.float32)] * 4
            + [pltpu.VMEM((C2, DA), jnp.float32),
               pltpu.VMEM((8, 16), jnp.float32),
               pltpu.VMEM_SHARED((N, DA), jnp.float32)]
            + [pltpu.SemaphoreType.DMA] * 5
        ),
    )


def _embed_body(x_ref, w_ref, b_ref, o_ref):
    o_ref[...] = jnp.dot(x_ref[...], w_ref[...],
                         preferred_element_type=jnp.float32) + b_ref[...]


def _tc_embed(x, w, b):
    return pl.pallas_call(
        _embed_body,
        out_shape=jax.ShapeDtypeStruct((N, D), jnp.float32),
    )(x, w, b.reshape(1, D))


def _lr_body(h_ref, wl_ref, bl_ref, wr_ref, br_ref, xl_ref, xr_ref):
    h = h_ref[...]
    xl = jnp.dot(h, wl_ref[...], preferred_element_type=jnp.float32) + bl_ref[...]
    xl_ref[:, :D] = xl
    unit = (lax.broadcasted_iota(jnp.int32, (1, 16), 1) == 0).astype(jnp.float32)
    xl_ref[:, D:] = jnp.broadcast_to(unit, (N, 16))
    xr_ref[...] = jnp.dot(h, wr_ref[...], preferred_element_type=jnp.float32) + br_ref[...]


def _tc_lr(h, wl, bl, wr, br):
    return pl.pallas_call(
        _lr_body,
        out_shape=[jax.ShapeDtypeStruct((N, DA), jnp.float32),
                   jax.ShapeDtypeStruct((N, D), jnp.float32)],
    )(h, wl, bl.reshape(1, D), wr, br.reshape(1, D))


_EW_BR = 2592


def _ew_body(a_ref, w_ref, o_ref):
    o_ref[...] = jnp.dot(a_ref[...], w_ref[...],
                         preferred_element_type=jnp.float32)


def _tc_ew(ea2p, we):
    return pl.pallas_call(
        _ew_body,
        grid=(TOTP // _EW_BR,),
        in_specs=[pl.BlockSpec((_EW_BR, ED), lambda i: (i, 0)),
                  pl.BlockSpec((ED, D), lambda i: (0, 0))],
        out_specs=pl.BlockSpec((_EW_BR, D), lambda i: (i, 0)),
        out_shape=jax.ShapeDtypeStruct((TOTP, D), jnp.float32),
    )(ea2p, we)


def _mean_body(a_ref, o_ref):
    s = a_ref[0, :N] + a_ref[1, :N]
    deg = jnp.maximum(s[:, ED:ED + 1], 1.0)
    o_ref[...] = s[:, :ED] / deg


def _tc_mean(acc1):
    return pl.pallas_call(
        _mean_body,
        out_shape=jax.ShapeDtypeStruct((N, ED), jnp.float32),
    )(acc1)


def _final_body(a_ref, b_ref, o_ref):
    s = a_ref[0] + a_ref[1]
    g = s[:, :D] / (s[:, D:D + 1] + 1e-16) + b_ref[...]
    o_ref[...] = g * jax.nn.sigmoid(g)


def _tc_final(acc2, bias):
    return pl.pallas_call(
        _final_body,
        out_shape=jax.ShapeDtypeStruct((N, D), jnp.float32),
    )(acc2, bias.reshape(1, D))


def kernel(x, edge_index, edge_attr, We_emb, be_emb,
           c1_Wl, c1_bl, c1_Wr, c1_br, c1_We, c1_att, c1_bias,
           c2_Wl, c2_bl, c2_Wr, c2_br, c2_We, c2_att, c2_bias):
    src = edge_index[0]
    dst = edge_index[1]

    d1 = dst.reshape(NW, NCH1, C1)
    eaa = jnp.concatenate(
        [edge_attr, jnp.ones((E, 1), jnp.float32), jnp.zeros((E, 15), jnp.float32)],
        axis=1)
    loop = jnp.arange(N, dtype=jnp.int32)
    pad = jnp.zeros((TOTP - TOT,), jnp.int32)
    s_all = jnp.concatenate([src, loop, pad]).reshape(NW, NCH2, C2)
    d_all = jnp.concatenate([dst, loop, pad]).reshape(NW, NCH2, C2)
    z32 = jnp.zeros((N_PAD, 2 * ED), jnp.float32)
    z144 = jnp.zeros((N, DA), jnp.float32)

    acc1 = _sc_degsum()(d1, eaa, z32)
    ea_mean = _tc_mean(acc1)
    ea2p = jnp.concatenate(
        [edge_attr, ea_mean, jnp.zeros((TOTP - TOT, ED), jnp.float32)], axis=0)

    h = _tc_embed(x, We_emb, be_emb)
    for (Wl, bl, Wr, br, We, att, bias) in (
            (c1_Wl, c1_bl, c1_Wr, c1_br, c1_We, c1_att, c1_bias),
            (c2_Wl, c2_bl, c2_Wr, c2_br, c2_We, c2_att, c2_bias)):
        ew = _tc_ew(ea2p, We)
        xl_aug, xr = _tc_lr(h, Wl, bl, Wr, br)
        acc2 = _sc_attn()(s_all, d_all, xl_aug, xr, ew, att.reshape(8, 16), z144)
        h = _tc_final(acc2, bias)
    return h


# speedup vs baseline: 7.6994x; 1.0037x over previous
import functools

import jax
import jax.numpy as jnp
from jax import lax
from jax.experimental import pallas as pl
from jax.experimental.pallas import tpu as pltpu
from jax.experimental.pallas import tpu_sc as plsc

N = 10000
E = 320000
D = 128
ED = 16
TOT = E + N
NC = 2
NS = 16
NW = NC * NS
N_PAD = 10240
ROWS_PER_TILE = N_PAD // NS

C1 = 80
NCH1 = (E // NW) // C1
M1 = E // NW

C2 = 48
NCH2 = 216
M2 = NCH2 * C2
TOTP = NW * M2
DA = D + 16


def _sc_degsum_body(d3, eaa, z32, out, d_idx, eab, acc, sem):
    cid = lax.axis_index("c")
    sid = lax.axis_index("s")
    wid = cid * NS + sid
    pltpu.sync_copy(d3.at[wid], d_idx)
    r0 = sid * ROWS_PER_TILE
    pltpu.sync_copy(z32.at[pl.ds(r0, ROWS_PER_TILE), :],
                    acc.at[pl.ds(r0, ROWS_PER_TILE), :])
    plsc.subcore_barrier()

    def chunk(j, carry):
        pltpu.async_copy(eaa.at[pl.ds(wid * M1 + j * C1, C1), :], eab, sem).wait()
        pltpu.sync_copy(eab, acc.at[d_idx.at[j]], add=True)
        return carry

    lax.fori_loop(0, NCH1, chunk, 0)
    plsc.subcore_barrier()
    pltpu.sync_copy(acc.at[pl.ds(r0, ROWS_PER_TILE), :],
                    out.at[cid, pl.ds(r0, ROWS_PER_TILE), :])


@functools.cache
def _sc_degsum():
    return pl.kernel(
        _sc_degsum_body,
        out_type=jax.ShapeDtypeStruct((NC, N_PAD, 2 * ED), jnp.float32),
        mesh=plsc.VectorSubcoreMesh(core_axis_name="c", subcore_axis_name="s",
                                    num_cores=NC, num_subcores=NS),
        compiler_params=pltpu.CompilerParams(needs_layout_passes=False, use_tc_tiling_on_sc=False),
        scratch_types=[
            pltpu.VMEM((NCH1, C1), jnp.int32),
            pltpu.VMEM((C1, 2 * ED), jnp.float32),
            pltpu.VMEM_SHARED((N_PAD, 2 * ED), jnp.float32),
            pltpu.SemaphoreType.DMA,
        ],
    )


def _sc_attn_body(sd3, xlt, xrt, ew, att8, z144, out,
                  sd00, sd01, sd10, sd11,
                  xlb0, xlb1, xrb0, xrb1, ewb0, ewb1,
                  attv, acc, semi0, semi1, semg0, semg1):
    cid = lax.axis_index("c")
    sid = lax.axis_index("s")
    wid = cid * NS + sid
    sd = ((sd00, sd01), (sd10, sd11))
    xlb = (xlb0, xlb1)
    xrb = (xrb0, xrb1)
    ewb = (ewb0, ewb1)
    semi = (semi0, semi1)
    semg = (semg0, semg1)
    RPT = N // NS
    pltpu.sync_copy(att8, attv)
    r0 = sid * RPT
    pltpu.sync_copy(z144.at[pl.ds(r0, RPT), :], acc.at[pl.ds(r0, RPT), :])
    plsc.subcore_barrier()

    def issue_idx(c, b, q):
        pltpu.async_copy(sd3.at[wid, c], sd[b][q], semi[b])

    def wait_idx(c, b, q):
        pltpu.make_async_copy(sd3.at[wid, c], sd[b][q], semi[b]).wait()

    def issue_gathers(c, b, q):
        pltpu.async_copy(xlt.at[sd[b][q].at[0]], xlb[b], semg[b])
        pltpu.async_copy(xrt.at[sd[b][q].at[1]], xrb[b], semg[b])
        pltpu.async_copy(ew.at[pl.ds(wid * M2 + c * C2, C2), :], ewb[b], semg[b])

    def wait_gathers(c, b, q):
        pltpu.make_async_copy(xlt.at[sd[b][q].at[0]], xlb[b], semg[b]).wait()
        pltpu.make_async_copy(xrt.at[sd[b][q].at[1]], xrb[b], semg[b]).wait()
        pltpu.make_async_copy(ew.at[pl.ds(wid * M2 + c * C2, C2), :],
                              ewb[b], semg[b]).wait()

    issue_idx(0, 0, 0)
    issue_idx(1, 1, 0)
    issue_idx(2, 0, 1)
    issue_idx(3, 1, 1)
    wait_idx(0, 0, 0)
    issue_gathers(0, 0, 0)
    wait_idx(1, 1, 0)
    issue_gathers(1, 1, 0)

    att_r = [attv[k, :] for k in range(8)]

    def quad(jq, carry):
        for m in range(4):
            b = m % 2
            q = m // 2
            c = jq * 4 + m
            wait_gathers(c, b, q)
            base_gid = wid * M2 + c * C2

            def group(g, gcarry):
                for u in range(8):
                    i = g * 8 + u
                    xl = [xlb[b][i, pl.ds(k * 16, 16)] for k in range(9)]
                    acc_v = None
                    for k in range(8):
                        v = (xl[k]
                             + xrb[b][i, pl.ds(k * 16, 16)]
                             + ewb[b][i, pl.ds(k * 16, 16)])
                        lv = jnp.maximum(v, 0.2 * v)
                        term = att_r[k] * lv
                        acc_v = term if acc_v is None else acc_v + term
                    tot = jnp.sum(acc_v)
                    ok = (base_gid + i) < TOT
                    tot = jnp.where(ok, tot, -1e30)
                    pvec = jnp.exp(jnp.full((16,), tot))
                    for k in range(9):
                        xlb[b][i, pl.ds(k * 16, 16)] = pvec * xl[k]
                return gcarry

            lax.fori_loop(0, C2 // 8, group, 0)
            pltpu.sync_copy(xlb[b], acc.at[sd[b][q].at[1]], add=True)

            @pl.when(c + 2 < NCH2)
            def _refill():
                wait_idx(c + 2, b, 1 - q)
                issue_gathers(c + 2, b, 1 - q)

                @pl.when(c + 4 < NCH2)
                def _prefetch_idx():
                    issue_idx(c + 4, b, q)
        return carry

    lax.fori_loop(0, NCH2 // 4, quad, 0)
    plsc.subcore_barrier()
    pltpu.sync_copy(acc.at[pl.ds(r0, RPT), :],
                    out.at[cid, pl.ds(r0, RPT), :])


@functools.cache
def _sc_attn():
    return pl.kernel(
        _sc_attn_body,
        out_type=jax.ShapeDtypeStruct((NC, N, DA), jnp.float32),
        mesh=plsc.VectorSubcoreMesh(core_axis_name="c", subcore_axis_name="s",
                                    num_cores=NC, num_subcores=NS),
        compiler_params=pltpu.CompilerParams(needs_layout_passes=False,
                                             use_tc_tiling_on_sc=False),
        scratch_types=(
            [pltpu.VMEM((2, C2), jnp.int32)] * 4
            + [pltpu.VMEM((C2, DA), jnp.float32)] * 2
            + [pltpu.VMEM((C2, D), jnp.float32)] * 4
            + [pltpu.VMEM((8, 16), jnp.float32),
               pltpu.VMEM_SHARED((N, DA), jnp.float32)]
            + [pltpu.SemaphoreType.DMA] * 4
        ),
    )


def _embed_body(x_ref, w_ref, b_ref, o_ref):
    o_ref[...] = jnp.dot(x_ref[...], w_ref[...],
                         preferred_element_type=jnp.float32) + b_ref[...]


def _tc_embed(x, w, b):
    return pl.pallas_call(
        _embed_body,
        out_shape=jax.ShapeDtypeStruct((N, D), jnp.float32),
    )(x, w, b.reshape(1, D))


def _lr_body(h_ref, wl_ref, bl_ref, wr_ref, br_ref, xl_ref, xr_ref):
    h = h_ref[...]
    xl = jnp.dot(h, wl_ref[...], preferred_element_type=jnp.float32) + bl_ref[...]
    xl_ref[:, :D] = xl
    unit = (lax.broadcasted_iota(jnp.int32, (1, 16), 1) == 0).astype(jnp.float32)
    xl_ref[:, D:] = jnp.broadcast_to(unit, (N, 16))
    xr_ref[...] = jnp.dot(h, wr_ref[...], preferred_element_type=jnp.float32) + br_ref[...]


def _tc_lr(h, wl, bl, wr, br):
    return pl.pallas_call(
        _lr_body,
        out_shape=[jax.ShapeDtypeStruct((N, DA), jnp.float32),
                   jax.ShapeDtypeStruct((N, D), jnp.float32)],
    )(h, wl, bl.reshape(1, D), wr, br.reshape(1, D))


_EW_BR = 2592


def _ew_body(a_ref, w_ref, o_ref):
    o_ref[...] = jnp.dot(a_ref[...], w_ref[...],
                         preferred_element_type=jnp.float32)


def _tc_ew(ea2p, we):
    return pl.pallas_call(
        _ew_body,
        grid=(TOTP // _EW_BR,),
        in_specs=[pl.BlockSpec((_EW_BR, ED), lambda i: (i, 0)),
                  pl.BlockSpec((ED, D), lambda i: (0, 0))],
        out_specs=pl.BlockSpec((_EW_BR, D), lambda i: (i, 0)),
        out_shape=jax.ShapeDtypeStruct((TOTP, D), jnp.float32),
    )(ea2p, we)


def _mean_body(a_ref, o_ref):
    s = a_ref[0, :N] + a_ref[1, :N]
    deg = jnp.maximum(s[:, ED:ED + 1], 1.0)
    o_ref[...] = s[:, :ED] / deg


def _tc_mean(acc1):
    return pl.pallas_call(
        _mean_body,
        out_shape=jax.ShapeDtypeStruct((N, ED), jnp.float32),
    )(acc1)


def _final_body(a_ref, b_ref, o_ref):
    s = a_ref[0] + a_ref[1]
    g = s[:, :D] / (s[:, D:D + 1] + 1e-16) + b_ref[...]
    o_ref[...] = g * jax.nn.sigmoid(g)


def _tc_final(acc2, bias):
    return pl.pallas_call(
        _final_body,
        out_shape=jax.ShapeDtypeStruct((N, D), jnp.float32),
    )(acc2, bias.reshape(1, D))


def kernel(x, edge_index, edge_attr, We_emb, be_emb,
           c1_Wl, c1_bl, c1_Wr, c1_br, c1_We, c1_att, c1_bias,
           c2_Wl, c2_bl, c2_Wr, c2_br, c2_We, c2_att, c2_bias):
    src = edge_index[0]
    dst = edge_index[1]

    d1 = dst.reshape(NW, NCH1, C1)
    eaa = jnp.concatenate(
        [edge_attr, jnp.ones((E, 1), jnp.float32), jnp.zeros((E, 15), jnp.float32)],
        axis=1)
    loop = jnp.arange(N, dtype=jnp.int32)
    pad = jnp.zeros((TOTP - TOT,), jnp.int32)
    s_all = jnp.concatenate([src, loop, pad]).reshape(NW, NCH2, 1, C2)
    d_all = jnp.concatenate([dst, loop, pad]).reshape(NW, NCH2, 1, C2)
    sd3 = jnp.concatenate([s_all, d_all], axis=2)
    z32 = jnp.zeros((N_PAD, 2 * ED), jnp.float32)
    z144 = jnp.zeros((N, DA), jnp.float32)

    acc1 = _sc_degsum()(d1, eaa, z32)
    ea_mean = _tc_mean(acc1)
    ea2p = jnp.concatenate(
        [edge_attr, ea_mean, jnp.zeros((TOTP - TOT, ED), jnp.float32)], axis=0)

    h = _tc_embed(x, We_emb, be_emb)
    for (Wl, bl, Wr, br, We, att, bias) in (
            (c1_Wl, c1_bl, c1_Wr, c1_br, c1_We, c1_att, c1_bias),
            (c2_Wl, c2_bl, c2_Wr, c2_br, c2_We, c2_att, c2_bias)):
        ew = _tc_ew(ea2p, We)
        xl_aug, xr = _tc_lr(h, Wl, bl, Wr, br)
        acc2 = _sc_attn()(sd3, xl_aug, xr, ew, att.reshape(8, 16), z144)
        h = _tc_final(acc2, bias)
    return h


# speedup vs baseline: 7.8283x; 1.0167x over previous
import functools

import jax
import jax.numpy as jnp
from jax import lax
from jax.experimental import pallas as pl
from jax.experimental.pallas import tpu as pltpu
from jax.experimental.pallas import tpu_sc as plsc

N = 10000
E = 320000
D = 128
ED = 16
TOT = E + N
NC = 2
NS = 16
NW = NC * NS
N_PAD = 10240
ROWS_PER_TILE = N_PAD // NS

C1 = 80
NCH1 = (E // NW) // C1
M1 = E // NW

C2 = 48
NCH2 = 216
M2 = NCH2 * C2
TOTP = NW * M2
DA = D + 16


def _sc_degsum_body(d3, eaa, z32, out, d_idx, eab, acc, sem):
    cid = lax.axis_index("c")
    sid = lax.axis_index("s")
    wid = cid * NS + sid
    pltpu.sync_copy(d3.at[wid], d_idx)
    r0 = sid * ROWS_PER_TILE
    pltpu.sync_copy(z32.at[pl.ds(r0, ROWS_PER_TILE), :],
                    acc.at[pl.ds(r0, ROWS_PER_TILE), :])
    plsc.subcore_barrier()

    def chunk(j, carry):
        pltpu.async_copy(eaa.at[pl.ds(wid * M1 + j * C1, C1), :], eab, sem).wait()
        pltpu.sync_copy(eab, acc.at[d_idx.at[j]], add=True)
        return carry

    lax.fori_loop(0, NCH1, chunk, 0)
    plsc.subcore_barrier()
    pltpu.sync_copy(acc.at[pl.ds(r0, ROWS_PER_TILE), :],
                    out.at[cid, pl.ds(r0, ROWS_PER_TILE), :])


@functools.cache
def _sc_degsum():
    return pl.kernel(
        _sc_degsum_body,
        out_type=jax.ShapeDtypeStruct((NC, N_PAD, 2 * ED), jnp.float32),
        mesh=plsc.VectorSubcoreMesh(core_axis_name="c", subcore_axis_name="s",
                                    num_cores=NC, num_subcores=NS),
        compiler_params=pltpu.CompilerParams(needs_layout_passes=False, use_tc_tiling_on_sc=False),
        scratch_types=[
            pltpu.VMEM((NCH1, C1), jnp.int32),
            pltpu.VMEM((C1, 2 * ED), jnp.float32),
            pltpu.VMEM_SHARED((N_PAD, 2 * ED), jnp.float32),
            pltpu.SemaphoreType.DMA,
        ],
    )


def _sc_attn_body(sd3, xlt, xrt, ew, att8, z144, out,
                  sd00, sd01, sd10, sd11,
                  xlb0, xlb1, xrb0, xrb1, ewb0, ewb1,
                  attv, acc, semi0, semi1, semg0, semg1):
    cid = lax.axis_index("c")
    sid = lax.axis_index("s")
    wid = cid * NS + sid
    sd = ((sd00, sd01), (sd10, sd11))
    xlb = (xlb0, xlb1)
    xrb = (xrb0, xrb1)
    ewb = (ewb0, ewb1)
    semi = (semi0, semi1)
    semg = (semg0, semg1)
    RPT = N // NS
    pltpu.sync_copy(att8, attv)
    r0 = sid * RPT
    pltpu.sync_copy(z144.at[pl.ds(r0, RPT), :], acc.at[pl.ds(r0, RPT), :])
    plsc.subcore_barrier()

    def issue_idx(c, b, q):
        pltpu.async_copy(sd3.at[wid, c], sd[b][q], semi[b])

    def wait_idx(c, b, q):
        pltpu.make_async_copy(sd3.at[wid, c], sd[b][q], semi[b]).wait()

    def issue_gathers(c, b, q):
        pltpu.async_copy(xlt.at[sd[b][q].at[0]], xlb[b], semg[b])
        pltpu.async_copy(xrt.at[sd[b][q].at[1]], xrb[b], semg[b])
        pltpu.async_copy(ew.at[pl.ds(wid * M2 + c * C2, C2), :], ewb[b], semg[b])

    def wait_gathers(c, b, q):
        pltpu.make_async_copy(xlt.at[sd[b][q].at[0]], xlb[b], semg[b]).wait()
        pltpu.make_async_copy(xrt.at[sd[b][q].at[1]], xrb[b], semg[b]).wait()
        pltpu.make_async_copy(ew.at[pl.ds(wid * M2 + c * C2, C2), :],
                              ewb[b], semg[b]).wait()

    issue_idx(0, 0, 0)
    issue_idx(1, 1, 0)
    issue_idx(2, 0, 1)
    issue_idx(3, 1, 1)
    wait_idx(0, 0, 0)
    issue_gathers(0, 0, 0)
    wait_idx(1, 1, 0)
    issue_gathers(1, 1, 0)

    att_r = [attv[k, :] for k in range(8)]

    def quad(jq, carry):
        for m in range(4):
            b = m % 2
            q = m // 2
            c = jq * 4 + m
            wait_gathers(c, b, q)
            base_gid = wid * M2 + c * C2

            @plsc.parallel_loop(0, C2 // 8)
            def group(g):
                for u in range(8):
                    i = g * 8 + u
                    xl = [xlb[b][i, pl.ds(k * 16, 16)] for k in range(9)]
                    acc_v = None
                    for k in range(8):
                        v = (xl[k]
                             + xrb[b][i, pl.ds(k * 16, 16)]
                             + ewb[b][i, pl.ds(k * 16, 16)])
                        lv = jnp.maximum(v, 0.2 * v)
                        term = att_r[k] * lv
                        acc_v = term if acc_v is None else acc_v + term
                    tot = jnp.sum(acc_v)
                    ok = (base_gid + i) < TOT
                    tot = jnp.where(ok, tot, -1e30)
                    pvec = jnp.exp(jnp.full((16,), tot))
                    for k in range(9):
                        xlb[b][i, pl.ds(k * 16, 16)] = pvec * xl[k]

            pltpu.sync_copy(xlb[b], acc.at[sd[b][q].at[1]], add=True)

            @pl.when(c + 2 < NCH2)
            def _refill():
                wait_idx(c + 2, b, 1 - q)
                issue_gathers(c + 2, b, 1 - q)

                @pl.when(c + 4 < NCH2)
                def _prefetch_idx():
                    issue_idx(c + 4, b, q)
        return carry

    lax.fori_loop(0, NCH2 // 4, quad, 0)
    plsc.subcore_barrier()
    pltpu.sync_copy(acc.at[pl.ds(r0, RPT), :],
                    out.at[cid, pl.ds(r0, RPT), :])


@functools.cache
def _sc_attn():
    return pl.kernel(
        _sc_attn_body,
        out_type=jax.ShapeDtypeStruct((NC, N, DA), jnp.float32),
        mesh=plsc.VectorSubcoreMesh(core_axis_name="c", subcore_axis_name="s",
                                    num_cores=NC, num_subcores=NS),
        compiler_params=pltpu.CompilerParams(needs_layout_passes=False,
                                             use_tc_tiling_on_sc=False),
        scratch_types=(
            [pltpu.VMEM((2, C2), jnp.int32)] * 4
            + [pltpu.VMEM((C2, DA), jnp.float32)] * 2
            + [pltpu.VMEM((C2, D), jnp.float32)] * 4
            + [pltpu.VMEM((8, 16), jnp.float32),
               pltpu.VMEM_SHARED((N, DA), jnp.float32)]
            + [pltpu.SemaphoreType.DMA] * 4
        ),
    )


def _embed_body(x_ref, w_ref, b_ref, o_ref):
    o_ref[...] = jnp.dot(x_ref[...], w_ref[...],
                         preferred_element_type=jnp.float32) + b_ref[...]


def _tc_embed(x, w, b):
    return pl.pallas_call(
        _embed_body,
        out_shape=jax.ShapeDtypeStruct((N, D), jnp.float32),
    )(x, w, b.reshape(1, D))


def _lr_body(h_ref, wl_ref, bl_ref, wr_ref, br_ref, xl_ref, xr_ref):
    h = h_ref[...]
    xl = jnp.dot(h, wl_ref[...], preferred_element_type=jnp.float32) + bl_ref[...]
    xl_ref[:, :D] = xl
    unit = (lax.broadcasted_iota(jnp.int32, (1, 16), 1) == 0).astype(jnp.float32)
    xl_ref[:, D:] = jnp.broadcast_to(unit, (N, 16))
    xr_ref[...] = jnp.dot(h, wr_ref[...], preferred_element_type=jnp.float32) + br_ref[...]


def _tc_lr(h, wl, bl, wr, br):
    return pl.pallas_call(
        _lr_body,
        out_shape=[jax.ShapeDtypeStruct((N, DA), jnp.float32),
                   jax.ShapeDtypeStruct((N, D), jnp.float32)],
    )(h, wl, bl.reshape(1, D), wr, br.reshape(1, D))


_EW_BR = 2592


def _ew_body(a_ref, w_ref, o_ref):
    o_ref[...] = jnp.dot(a_ref[...], w_ref[...],
                         preferred_element_type=jnp.float32)


def _tc_ew(ea2p, we):
    return pl.pallas_call(
        _ew_body,
        grid=(TOTP // _EW_BR,),
        in_specs=[pl.BlockSpec((_EW_BR, ED), lambda i: (i, 0)),
                  pl.BlockSpec((ED, D), lambda i: (0, 0))],
        out_specs=pl.BlockSpec((_EW_BR, D), lambda i: (i, 0)),
        out_shape=jax.ShapeDtypeStruct((TOTP, D), jnp.float32),
    )(ea2p, we)


def _mean_body(a_ref, o_ref):
    s = a_ref[0, :N] + a_ref[1, :N]
    deg = jnp.maximum(s[:, ED:ED + 1], 1.0)
    o_ref[...] = s[:, :ED] / deg


def _tc_mean(acc1):
    return pl.pallas_call(
        _mean_body,
        out_shape=jax.ShapeDtypeStruct((N, ED), jnp.float32),
    )(acc1)


def _final_body(a_ref, b_ref, o_ref):
    s = a_ref[0] + a_ref[1]
    g = s[:, :D] / (s[:, D:D + 1] + 1e-16) + b_ref[...]
    o_ref[...] = g * jax.nn.sigmoid(g)


def _tc_final(acc2, bias):
    return pl.pallas_call(
        _final_body,
        out_shape=jax.ShapeDtypeStruct((N, D), jnp.float32),
    )(acc2, bias.reshape(1, D))


def kernel(x, edge_index, edge_attr, We_emb, be_emb,
           c1_Wl, c1_bl, c1_Wr, c1_br, c1_We, c1_att, c1_bias,
           c2_Wl, c2_bl, c2_Wr, c2_br, c2_We, c2_att, c2_bias):
    src = edge_index[0]
    dst = edge_index[1]

    d1 = dst.reshape(NW, NCH1, C1)
    eaa = jnp.concatenate(
        [edge_attr, jnp.ones((E, 1), jnp.float32), jnp.zeros((E, 15), jnp.float32)],
        axis=1)
    loop = jnp.arange(N, dtype=jnp.int32)
    pad = jnp.zeros((TOTP - TOT,), jnp.int32)
    s_all = jnp.concatenate([src, loop, pad]).reshape(NW, NCH2, 1, C2)
    d_all = jnp.concatenate([dst, loop, pad]).reshape(NW, NCH2, 1, C2)
    sd3 = jnp.concatenate([s_all, d_all], axis=2)
    z32 = jnp.zeros((N_PAD, 2 * ED), jnp.float32)
    z144 = jnp.zeros((N, DA), jnp.float32)

    acc1 = _sc_degsum()(d1, eaa, z32)
    ea_mean = _tc_mean(acc1)
    ea2p = jnp.concatenate(
        [edge_attr, ea_mean, jnp.zeros((TOTP - TOT, ED), jnp.float32)], axis=0)

    h = _tc_embed(x, We_emb, be_emb)
    for (Wl, bl, Wr, br, We, att, bias) in (
            (c1_Wl, c1_bl, c1_Wr, c1_br, c1_We, c1_att, c1_bias),
            (c2_Wl, c2_bl, c2_Wr, c2_br, c2_We, c2_att, c2_bias)):
        ew = _tc_ew(ea2p, We)
        xl_aug, xr = _tc_lr(h, Wl, bl, Wr, br)
        acc2 = _sc_attn()(sd3, xl_aug, xr, ew, att.reshape(8, 16), z144)
        h = _tc_final(acc2, bias)
    return h


# speedup vs baseline: 7.8682x; 1.0051x over previous
import functools

import jax
import jax.numpy as jnp
from jax import lax
from jax.experimental import pallas as pl
from jax.experimental.pallas import tpu as pltpu
from jax.experimental.pallas import tpu_sc as plsc

N = 10000
E = 320000
D = 128
ED = 16
TOT = E + N
NC = 2
NS = 16
NW = NC * NS
N_PAD = 10240
ROWS_PER_TILE = N_PAD // NS

C1 = 40
NCH1 = (E // NW) // C1
M1 = E // NW

C2 = 48
NCH2 = 216
M2 = NCH2 * C2
TOTP = NW * M2
DA = D + 16


def _sc_degsum_body(d3, eaa, z32, out, d_idx, eab0, eab1, acc, sem0, sem1):
    cid = lax.axis_index("c")
    sid = lax.axis_index("s")
    wid = cid * NS + sid
    eab = (eab0, eab1)
    sem = (sem0, sem1)
    pltpu.sync_copy(d3.at[wid], d_idx)
    r0 = sid * ROWS_PER_TILE
    pltpu.sync_copy(z32.at[pl.ds(r0, ROWS_PER_TILE), :],
                    acc.at[pl.ds(r0, ROWS_PER_TILE), :])
    plsc.subcore_barrier()

    def issue(j, b):
        pltpu.async_copy(eaa.at[pl.ds(wid * M1 + j * C1, C1), :], eab[b], sem[b])

    def wait(j, b):
        pltpu.make_async_copy(eaa.at[pl.ds(wid * M1 + j * C1, C1), :],
                              eab[b], sem[b]).wait()

    issue(0, 0)
    issue(1, 1)

    def pair(jp, carry):
        for b in range(2):
            j = jp * 2 + b
            wait(j, b)
            pltpu.sync_copy(eab[b], acc.at[d_idx.at[j]], add=True)

            @pl.when(j + 2 < NCH1)
            def _refill():
                issue(j + 2, b)
        return carry

    lax.fori_loop(0, NCH1 // 2, pair, 0)
    plsc.subcore_barrier()
    pltpu.sync_copy(acc.at[pl.ds(r0, ROWS_PER_TILE), :],
                    out.at[cid, pl.ds(r0, ROWS_PER_TILE), :])


@functools.cache
def _sc_degsum():
    return pl.kernel(
        _sc_degsum_body,
        out_type=jax.ShapeDtypeStruct((NC, N_PAD, 2 * ED), jnp.float32),
        mesh=plsc.VectorSubcoreMesh(core_axis_name="c", subcore_axis_name="s",
                                    num_cores=NC, num_subcores=NS),
        compiler_params=pltpu.CompilerParams(needs_layout_passes=False, use_tc_tiling_on_sc=False),
        scratch_types=[
            pltpu.VMEM((NCH1, C1), jnp.int32),
            pltpu.VMEM((C1, 2 * ED), jnp.float32),
            pltpu.VMEM((C1, 2 * ED), jnp.float32),
            pltpu.VMEM_SHARED((N_PAD, 2 * ED), jnp.float32),
            pltpu.SemaphoreType.DMA,
            pltpu.SemaphoreType.DMA,
        ],
    )


def _sc_attn_body(sd3, xlt, xrt, ew, att8, z144, out,
                  sd00, sd01, sd10, sd11,
                  xlb0, xlb1, xrb0, xrb1, ewb0, ewb1,
                  attv, acc, semi0, semi1, semg0, semg1):
    cid = lax.axis_index("c")
    sid = lax.axis_index("s")
    wid = cid * NS + sid
    sd = ((sd00, sd01), (sd10, sd11))
    xlb = (xlb0, xlb1)
    xrb = (xrb0, xrb1)
    ewb = (ewb0, ewb1)
    semi = (semi0, semi1)
    semg = (semg0, semg1)
    RPT = N // NS
    pltpu.sync_copy(att8, attv)
    r0 = sid * RPT
    pltpu.sync_copy(z144.at[pl.ds(r0, RPT), :], acc.at[pl.ds(r0, RPT), :])
    plsc.subcore_barrier()

    def issue_idx(c, b, q):
        pltpu.async_copy(sd3.at[wid, c], sd[b][q], semi[b])

    def wait_idx(c, b, q):
        pltpu.make_async_copy(sd3.at[wid, c], sd[b][q], semi[b]).wait()

    def issue_gathers(c, b, q):
        pltpu.async_copy(xlt.at[sd[b][q].at[0]], xlb[b], semg[b])
        pltpu.async_copy(xrt.at[sd[b][q].at[1]], xrb[b], semg[b])
        pltpu.async_copy(ew.at[pl.ds(wid * M2 + c * C2, C2), :], ewb[b], semg[b])

    def wait_gathers(c, b, q):
        pltpu.make_async_copy(xlt.at[sd[b][q].at[0]], xlb[b], semg[b]).wait()
        pltpu.make_async_copy(xrt.at[sd[b][q].at[1]], xrb[b], semg[b]).wait()
        pltpu.make_async_copy(ew.at[pl.ds(wid * M2 + c * C2, C2), :],
                              ewb[b], semg[b]).wait()

    issue_idx(0, 0, 0)
    issue_idx(1, 1, 0)
    issue_idx(2, 0, 1)
    issue_idx(3, 1, 1)
    wait_idx(0, 0, 0)
    issue_gathers(0, 0, 0)
    wait_idx(1, 1, 0)
    issue_gathers(1, 1, 0)

    att_r = [attv[k, :] for k in range(8)]

    def quad(jq, carry):
        for m in range(4):
            b = m % 2
            q = m // 2
            c = jq * 4 + m
            wait_gathers(c, b, q)
            base_gid = wid * M2 + c * C2

            @plsc.parallel_loop(0, C2 // 8)
            def group(g):
                for u in range(8):
                    i = g * 8 + u
                    xl = [xlb[b][i, pl.ds(k * 16, 16)] for k in range(9)]
                    acc_v = None
                    for k in range(8):
                        v = (xl[k]
                             + xrb[b][i, pl.ds(k * 16, 16)]
                             + ewb[b][i, pl.ds(k * 16, 16)])
                        lv = jnp.maximum(v, 0.2 * v)
                        term = att_r[k] * lv
                        acc_v = term if acc_v is None else acc_v + term
                    tot = jnp.sum(acc_v)
                    ok = (base_gid + i) < TOT
                    tot = jnp.where(ok, tot, -1e30)
                    pvec = jnp.exp(jnp.full((16,), tot))
                    for k in range(9):
                        xlb[b][i, pl.ds(k * 16, 16)] = pvec * xl[k]

            pltpu.sync_copy(xlb[b], acc.at[sd[b][q].at[1]], add=True)

            @pl.when(c + 2 < NCH2)
            def _refill():
                wait_idx(c + 2, b, 1 - q)
                issue_gathers(c + 2, b, 1 - q)

                @pl.when(c + 4 < NCH2)
                def _prefetch_idx():
                    issue_idx(c + 4, b, q)
        return carry

    lax.fori_loop(0, NCH2 // 4, quad, 0)
    plsc.subcore_barrier()
    pltpu.sync_copy(acc.at[pl.ds(r0, RPT), :],
                    out.at[cid, pl.ds(r0, RPT), :])


@functools.cache
def _sc_attn():
    return pl.kernel(
        _sc_attn_body,
        out_type=jax.ShapeDtypeStruct((NC, N, DA), jnp.float32),
        mesh=plsc.VectorSubcoreMesh(core_axis_name="c", subcore_axis_name="s",
                                    num_cores=NC, num_subcores=NS),
        compiler_params=pltpu.CompilerParams(needs_layout_passes=False,
                                             use_tc_tiling_on_sc=False),
        scratch_types=(
            [pltpu.VMEM((2, C2), jnp.int32)] * 4
            + [pltpu.VMEM((C2, DA), jnp.float32)] * 2
            + [pltpu.VMEM((C2, D), jnp.float32)] * 4
            + [pltpu.VMEM((8, 16), jnp.float32),
               pltpu.VMEM_SHARED((N, DA), jnp.float32)]
            + [pltpu.SemaphoreType.DMA] * 4
        ),
    )


def _embed_body(x_ref, w_ref, b_ref, o_ref):
    o_ref[...] = jnp.dot(x_ref[...], w_ref[...],
                         preferred_element_type=jnp.float32) + b_ref[...]


def _tc_embed(x, w, b):
    return pl.pallas_call(
        _embed_body,
        out_shape=jax.ShapeDtypeStruct((N, D), jnp.float32),
    )(x, w, b.reshape(1, D))


def _lr_body(h_ref, wl_ref, bl_ref, wr_ref, br_ref, xl_ref, xr_ref):
    h = h_ref[...]
    xl = jnp.dot(h, wl_ref[...], preferred_element_type=jnp.float32) + bl_ref[...]
    xl_ref[:, :D] = xl
    unit = (lax.broadcasted_iota(jnp.int32, (1, 16), 1) == 0).astype(jnp.float32)
    xl_ref[:, D:] = jnp.broadcast_to(unit, (N, 16))
    xr_ref[...] = jnp.dot(h, wr_ref[...], preferred_element_type=jnp.float32) + br_ref[...]


def _tc_lr(h, wl, bl, wr, br):
    return pl.pallas_call(
        _lr_body,
        out_shape=[jax.ShapeDtypeStruct((N, DA), jnp.float32),
                   jax.ShapeDtypeStruct((N, D), jnp.float32)],
    )(h, wl, bl.reshape(1, D), wr, br.reshape(1, D))


_EW_BR = 2592


def _ew_body(a_ref, w_ref, o_ref):
    o_ref[...] = jnp.dot(a_ref[...], w_ref[...],
                         preferred_element_type=jnp.float32)


def _tc_ew(ea2p, we):
    return pl.pallas_call(
        _ew_body,
        grid=(TOTP // _EW_BR,),
        in_specs=[pl.BlockSpec((_EW_BR, ED), lambda i: (i, 0)),
                  pl.BlockSpec((ED, D), lambda i: (0, 0))],
        out_specs=pl.BlockSpec((_EW_BR, D), lambda i: (i, 0)),
        out_shape=jax.ShapeDtypeStruct((TOTP, D), jnp.float32),
    )(ea2p, we)


def _mean_body(a_ref, o_ref):
    s = a_ref[0, :N] + a_ref[1, :N]
    deg = jnp.maximum(s[:, ED:ED + 1], 1.0)
    o_ref[...] = s[:, :ED] / deg


def _tc_mean(acc1):
    return pl.pallas_call(
        _mean_body,
        out_shape=jax.ShapeDtypeStruct((N, ED), jnp.float32),
    )(acc1)


def _final_body(a_ref, b_ref, o_ref):
    s = a_ref[0] + a_ref[1]
    g = s[:, :D] / (s[:, D:D + 1] + 1e-16) + b_ref[...]
    o_ref[...] = g * jax.nn.sigmoid(g)


def _tc_final(acc2, bias):
    return pl.pallas_call(
        _final_body,
        out_shape=jax.ShapeDtypeStruct((N, D), jnp.float32),
    )(acc2, bias.reshape(1, D))


def kernel(x, edge_index, edge_attr, We_emb, be_emb,
           c1_Wl, c1_bl, c1_Wr, c1_br, c1_We, c1_att, c1_bias,
           c2_Wl, c2_bl, c2_Wr, c2_br, c2_We, c2_att, c2_bias):
    src = edge_index[0]
    dst = edge_index[1]

    d1 = dst.reshape(NW, NCH1, C1)
    eaa = jnp.concatenate(
        [edge_attr, jnp.ones((E, 1), jnp.float32), jnp.zeros((E, 15), jnp.float32)],
        axis=1)
    loop = jnp.arange(N, dtype=jnp.int32)
    pad = jnp.zeros((TOTP - TOT,), jnp.int32)
    s_all = jnp.concatenate([src, loop, pad]).reshape(NW, NCH2, 1, C2)
    d_all = jnp.concatenate([dst, loop, pad]).reshape(NW, NCH2, 1, C2)
    sd3 = jnp.concatenate([s_all, d_all], axis=2)
    z32 = jnp.zeros((N_PAD, 2 * ED), jnp.float32)
    z144 = jnp.zeros((N, DA), jnp.float32)

    acc1 = _sc_degsum()(d1, eaa, z32)
    ea_mean = _tc_mean(acc1)
    ea2p = jnp.concatenate(
        [edge_attr, ea_mean, jnp.zeros((TOTP - TOT, ED), jnp.float32)], axis=0)

    h = _tc_embed(x, We_emb, be_emb)
    for (Wl, bl, Wr, br, We, att, bias) in (
            (c1_Wl, c1_bl, c1_Wr, c1_br, c1_We, c1_att, c1_bias),
            (c2_Wl, c2_bl, c2_Wr, c2_br, c2_We, c2_att, c2_bias)):
        ew = _tc_ew(ea2p, We)
        xl_aug, xr = _tc_lr(h, Wl, bl, Wr, br)
        acc2 = _sc_attn()(sd3, xl_aug, xr, ew, att.reshape(8, 16), z144)
        h = _tc_final(acc2, bias)
    return h
